# baseline (speedup 1.0000x reference)
import functools

import jax
import jax.numpy as jnp
from jax import lax
from jax.experimental import pallas as pl
from jax.experimental.pallas import tpu as pltpu
from jax.experimental.pallas import tpu_sc as plsc

_TOP_K_PER_ROW = 8

_INT_MIN = -(2**31)
_MONO_XOR = 0x7FFFFFFF


def _gate_body(x_ref, w1_ref, b1_ref, w2_ref, b2_ref, keys_ref, mkey_ref,
               a_ref):
    h2 = pl.program_id(0)
    s = pl.program_id(1)
    bs = x_ref.shape[0]

    @pl.when(h2 == 0)
    def _():
        a = lax.dot_general(x_ref[...], w1_ref[...], (((1,), (1,)), ((), ())),
                            preferred_element_type=jnp.float32)
        a_ref[pl.ds(s * bs, bs), :] = jnp.maximum(a + b1_ref[...], 0.0)

    sc = lax.dot_general(a_ref[pl.ds(s * bs, bs), :], w2_ref[...],
                         (((1,), (1,)), ((), ())),
                         preferred_element_type=jnp.float32)
    sc = sc + b2_ref[...]
    bits = lax.bitcast_convert_type(sc, jnp.int32)
    skey = jnp.where(bits >= 0, bits, bits ^ jnp.int32(_MONO_XOR))
    keys_ref[...] = skey
    cm = jnp.max(skey, axis=0, keepdims=True)
    prev = jnp.where(s == 0, jnp.int32(_INT_MIN), mkey_ref[...])
    mkey_ref[...] = jnp.maximum(prev, cm)


def _gate_keys(x, w1, b1_row, w2, b2_row, bs=256, bh=512):
    S, D = x.shape
    H = w1.shape[0]
    return pl.pallas_call(
        _gate_body,
        grid=(H // bh, S // bs),
        in_specs=[
            pl.BlockSpec((bs, D), lambda h, s: (s, 0)),
            pl.BlockSpec((H, D), lambda h, s: (0, 0)),
            pl.BlockSpec((1, H), lambda h, s: (0, 0)),
            pl.BlockSpec((bh, H), lambda h, s: (h, 0)),
            pl.BlockSpec((1, bh), lambda h, s: (0, h)),
        ],
        out_specs=[
            pl.BlockSpec((bs, bh), lambda h, s: (s, h)),
            pl.BlockSpec((1, bh), lambda h, s: (0, h)),
        ],
        out_shape=[
            jax.ShapeDtypeStruct((S, H), jnp.int32),
            jax.ShapeDtypeStruct((1, H), jnp.int32),
        ],
        scratch_shapes=[pltpu.VMEM((S, H), jnp.float32)],
    )(x, w1, b1_row, w2, b2_row)


_NLANES = 16
_WIN = 32768
_MROWS = 8
_UNROLL = 8


def _hist_field(keys_hbm, win_v, sem0, sem1, lh_v, base, shift_bin, binmask,
                bin_off, shift_pred, pred_val, nbins, nwin):
    lanes = lax.iota(jnp.int32, _NLANES)
    lane_off = lanes * nbins
    ones = jnp.ones((_NLANES,), jnp.int32)

    zero = jnp.zeros((_NLANES,), jnp.int32)

    def zbody(j, _):
        for u in range(4):
            lh_v[pl.ds((j * 4 + u) * _NLANES, _NLANES)] = zero
        return 0
    lax.fori_loop(0, nbins // 4, zbody, 0)

    sems = [sem0, sem1]
    copies = [None, None]

    def start(w):
        b = w % 2
        copies[b] = pltpu.async_copy(
            keys_hbm.at[pl.ds(base + w * _WIN, _WIN)],
            win_v.at[pl.ds(b * _WIN, _WIN)], sems[b])

    def process(w):
        boff = (w % 2) * _WIN

        def ibody(i, _):
            ib = boff + i * (_NLANES * _UNROLL)
            vs = [win_v[pl.ds(ib + u * _NLANES, _NLANES)]
                  for u in range(_UNROLL)]
            for v in vs:
                bv = ((v >> shift_bin) & binmask) + bin_off
                addr = lane_off + bv
                if shift_pred is None:
                    plsc.addupdate_scatter(lh_v, [addr], ones)
                else:
                    pred = (v >> shift_pred) == pred_val
                    plsc.addupdate_scatter(lh_v, [addr], ones, mask=pred)
            return 0
        lax.fori_loop(0, _WIN // (_NLANES * _UNROLL), ibody, 0)

    start(0)
    for w in range(nwin):
        if w + 1 < nwin:
            start(w + 1)
        copies[w % 2].wait()
        process(w)


def _merge_lanes(lh_v, mh_v, nbins):
    def mbody(j, _):
        acc = jnp.zeros((_NLANES,), jnp.int32)
        for lane in range(_NLANES):
            acc = acc + lh_v[pl.ds(lane * nbins + j * _NLANES, _NLANES)]
        mh_v[pl.ds(j * _NLANES, _NLANES)] = acc
        return 0
    lax.fori_loop(0, nbins // _NLANES, mbody, 0)


def _radix_select(keys_flat, k):
    N = keys_flat.shape[0]
    nc, ns = 2, 16
    chunk = N // ns
    nwin = chunk // _WIN
    assert chunk % _WIN == 0 and N % ns == 0
    mesh = plsc.VectorSubcoreMesh(core_axis_name="c", subcore_axis_name="s",
                                  num_cores=nc, num_subcores=ns)

    @functools.partial(
        pl.kernel,
        out_type=[jax.ShapeDtypeStruct((1, 1024), jnp.int32),
                  jax.ShapeDtypeStruct((16,), jnp.int32)],
        mesh=mesh,
        compiler_params=pltpu.CompilerParams(needs_layout_passes=False),
        scratch_types=[
            pltpu.VMEM((2 * _WIN,), jnp.int32),
            pltpu.SemaphoreType.DMA,
            pltpu.SemaphoreType.DMA,
            pltpu.VMEM((_NLANES * 2048,), jnp.int32),
            pltpu.VMEM((2048,), jnp.int32),
            pltpu.VMEM((2048,), jnp.int32),
            pltpu.VMEM_SHARED((_NLANES, 2048), jnp.int32),
        ],
    )
    def sel(keys_hbm, h3_hbm, st_hbm, win_v, sem0, sem1, lh_v, mh_v, row_v,
            stage_sh):
        cid = lax.axis_index("c")
        sid = lax.axis_index("s")
        base = sid * chunk

        def merge_scan_local(nbins, p_prev, c_hi):
            _merge_lanes(lh_v, mh_v, nbins)
            pltpu.sync_copy(mh_v.at[pl.ds(0, nbins)],
                            stage_sh.at[sid, pl.ds(0, nbins)])
            plsc.subcore_barrier()
            nchunk = nbins // _NLANES

            def zb(j, _):
                mh_v[pl.ds(j * _NLANES, _NLANES)] = jnp.zeros(
                    (_NLANES,), jnp.int32)
                return 0
            lax.fori_loop(0, nchunk, zb, 0)

            def rb(t, _):
                pltpu.sync_copy(stage_sh.at[t, pl.ds(0, nbins)],
                                row_v.at[pl.ds(0, nbins)])

                def ab(j, _):
                    sl = pl.ds(j * _NLANES, _NLANES)
                    mh_v[sl] = mh_v[sl] + row_v[sl]
                    return 0
                lax.fori_loop(0, nchunk, ab, 0)
                return 0
            lax.fori_loop(0, _NLANES, rb, 0)
            plsc.subcore_barrier()

            kk = jnp.int32(k)

            def cb(j, carry):
                cum, csel, cums = carry
                c = jnp.int32(nchunk - 1) - j
                vchunk = mh_v[pl.ds(c * _NLANES, _NLANES)]
                ssum = jnp.sum(vchunk, axis=0)
                hit = jnp.logical_and(cum + ssum >= kk, csel < 0)
                csel = jnp.where(hit, c, csel)
                cums = jnp.where(hit, cum, cums)
                return (cum + ssum, csel, cums)

            _, csel, cum0 = lax.fori_loop(
                0, nchunk, cb, (c_hi, jnp.int32(-1), jnp.int32(0)))

            vchunk = mh_v[pl.ds(csel * _NLANES, _NLANES)]
            lanes = lax.iota(jnp.int32, _NLANES)
            csum = plsc.cumsum(vchunk)
            total = jnp.sum(vchunk, axis=0)
            s_incl = cum0 + (total - csum) + vchunk
            hit = s_incl >= kk
            nhits = plsc.all_reduce_population_count(hit)
            bl = nhits[0] - 1
            bsel = csel * _NLANES + bl
            above = jnp.where(lanes > bl, vchunk, 0)
            chi = cum0 + jnp.sum(above, axis=0)
            return bsel, chi

        _hist_field(keys_hbm, win_v, sem0, sem1, lh_v, base,
                    jnp.int32(21), jnp.int32(2047), jnp.int32(1024),
                    None, None, 2048, nwin)
        bsel, chi1 = merge_scan_local(2048, jnp.int32(0), jnp.int32(0))
        p1 = bsel - jnp.int32(1024)

        _hist_field(keys_hbm, win_v, sem0, sem1, lh_v, base,
                    jnp.int32(10), jnp.int32(2047), jnp.int32(0),
                    jnp.int32(21), p1, 2048, nwin)
        bsel2, chi2 = merge_scan_local(2048, p1, chi1)
        p2 = p1 * jnp.int32(2048) + bsel2

        _hist_field(keys_hbm, win_v, sem0, sem1, lh_v, base,
                    jnp.int32(0), jnp.int32(1023), jnp.int32(0),
                    jnp.int32(10), p2, 1024, nwin)
        _merge_lanes(lh_v, mh_v, 1024)
        pltpu.sync_copy(mh_v.at[pl.ds(0, 1024)],
                        stage_sh.at[sid, pl.ds(0, 1024)])
        plsc.subcore_barrier()

        @pl.when(jnp.logical_and(cid == 0, sid == 0))
        def _():
            def zb(j, _):
                mh_v[pl.ds(j * _NLANES, _NLANES)] = jnp.zeros(
                    (_NLANES,), jnp.int32)
                return 0
            lax.fori_loop(0, 64, zb, 0)

            def rb(t, _):
                pltpu.sync_copy(stage_sh.at[t, pl.ds(0, 1024)],
                                row_v.at[pl.ds(0, 1024)])

                def ab(j, _):
                    sl = pl.ds(j * _NLANES, _NLANES)
                    mh_v[sl] = mh_v[sl] + row_v[sl]
                    return 0
                lax.fori_loop(0, 64, ab, 0)
                return 0
            lax.fori_loop(0, _NLANES, rb, 0)
            pltpu.sync_copy(mh_v.at[pl.ds(0, 1024)], h3_hbm.at[0])
            st = jnp.where(lax.iota(jnp.int32, 16) == 0, p2,
                           jnp.where(lax.iota(jnp.int32, 16) == 1, chi2, 0))
            row_v[pl.ds(0, 16)] = st
            pltpu.sync_copy(row_v.at[pl.ds(0, 16)], st_hbm)

    h3, st2 = sel(keys_flat)
    return h3, st2


def _wsel_body(up_ref, mod_ref, mcol_ref, h3_ref, st_ref, k_ref, w_ref,
               thr_ref):
    nb = h3_ref.shape[1]
    hist = jnp.sum(h3_ref[...].astype(jnp.float32), axis=0, keepdims=True)
    bi = lax.broadcasted_iota(jnp.int32, (nb, nb), 0)
    bj = lax.broadcasted_iota(jnp.int32, (nb, nb), 1)
    lower = jnp.where(bi >= bj, 1.0, 0.0)
    suffix = lax.dot_general(hist, lower, (((1,), (0,)), ((), ())),
                             preferred_element_type=jnp.float32)
    stv = st_ref[...]
    p2 = stv[0, 0]
    c_hi = stv[0, 1].astype(jnp.float32)
    kk = k_ref[...][0, 0].astype(jnp.float32)
    hits = jnp.sum(jnp.where(suffix + c_hi >= kk, 1, 0))
    bsel = hits.astype(jnp.int32) - 1
    thr = p2 * jnp.int32(nb) + bsel
    thr_ref[...] = jnp.zeros_like(thr_ref) + thr
    mask = mcol_ref[...] >= thr
    w_ref[...] = jnp.where(mask, mod_ref[...], up_ref[...])


def _wsel(up_w, mod_w, mkey_col, h3, st2, k, bh=384):
    H, D = up_w.shape
    nw, nb = h3.shape
    karr = jnp.full((1, 16), k, jnp.int32)
    return pl.pallas_call(
        _wsel_body,
        grid=(H // bh,),
        in_specs=[
            pl.BlockSpec((bh, D), lambda h: (h, 0)),
            pl.BlockSpec((bh, D), lambda h: (h, 0)),
            pl.BlockSpec((bh, 1), lambda h: (h, 0)),
            pl.BlockSpec((nw, nb), lambda h: (0, 0)),
            pl.BlockSpec((1, 16), lambda h: (0, 0)),
            pl.BlockSpec((1, 16), lambda h: (0, 0)),
        ],
        out_specs=[
            pl.BlockSpec((bh, D), lambda h: (h, 0)),
            pl.BlockSpec((1, 16), lambda h: (0, 0)),
        ],
        out_shape=[
            jax.ShapeDtypeStruct((H, D), jnp.float32),
            jax.ShapeDtypeStruct((1, 16), jnp.int32),
        ],
    )(up_w, mod_w, mkey_col, h3, st2.reshape(1, 16), karr)


def _final_body(x_ref, w_ref, dw_ref, mrow_ref, thr_ref, upb_ref, modb_ref,
                db_ref, o_ref):
    h = lax.dot_general(x_ref[...], w_ref[...], (((1,), (1,)), ((), ())),
                        preferred_element_type=jnp.float32)
    bsel = jnp.where(mrow_ref[...] >= thr_ref[...][0, 0], modb_ref[...],
                     upb_ref[...])
    h = h + bsel
    g = 0.5 * h * (1.0 + lax.erf(h * 0.7071067811865476))
    out = lax.dot_general(g, dw_ref[...], (((1,), (1,)), ((), ())),
                          preferred_element_type=jnp.float32)
    o_ref[...] = out + db_ref[...]


def _final(x, w_sel, down_w, mkey_row, thr_row, up_b_row, mod_b_row,
           down_b_row, bs=256):
    S, D = x.shape
    H = w_sel.shape[0]
    return pl.pallas_call(
        _final_body,
        grid=(S // bs,),
        in_specs=[
            pl.BlockSpec((bs, D), lambda s: (s, 0)),
            pl.BlockSpec((H, D), lambda s: (0, 0)),
            pl.BlockSpec((D, H), lambda s: (0, 0)),
            pl.BlockSpec((1, H), lambda s: (0, 0)),
            pl.BlockSpec((1, 16), lambda s: (0, 0)),
            pl.BlockSpec((1, H), lambda s: (0, 0)),
            pl.BlockSpec((1, H), lambda s: (0, 0)),
            pl.BlockSpec((1, D), lambda s: (0, 0)),
        ],
        out_specs=pl.BlockSpec((bs, D), lambda s: (s, 0)),
        out_shape=jax.ShapeDtypeStruct((S, D), jnp.float32),
    )(x, w_sel, down_w, mkey_row, thr_row, up_b_row, mod_b_row, down_b_row)


def kernel(inputs, up_w, up_b, gate_w1, gate_b1, gate_w2, gate_b2,
           mod_w, mod_b, down_w, down_b):
    B, S, D = inputs.shape
    H = up_w.shape[0]
    k = min(_TOP_K_PER_ROW * S, S * H)
    x = inputs.reshape(S, D)

    keys, mkey = _gate_keys(x, gate_w1, gate_b1.reshape(1, H), gate_w2,
                            gate_b2.reshape(1, H))
    h3, st2 = _radix_select(keys.reshape(S * H), k)
    h3m = jnp.sum(h3, axis=0, keepdims=True)
    w_sel, thr_row = _wsel(up_w, mod_w, mkey.reshape(H, 1), h3m, st2, k)
    out = _final(x, w_sel, down_w, mkey, thr_row, up_b.reshape(1, H),
                 mod_b.reshape(1, H), down_b.reshape(1, D))
    return out.reshape(B, S, D)


# speedup vs baseline: 1.1552x; 1.1552x over previous
import functools

import jax
import jax.numpy as jnp
from jax import lax
from jax.experimental import pallas as pl
from jax.experimental.pallas import tpu as pltpu
from jax.experimental.pallas import tpu_sc as plsc

_TOP_K_PER_ROW = 8

_INT_MIN = -(2**31)
_MONO_XOR = 0x7FFFFFFF


def _gate_body(x_ref, w1_ref, b1_ref, w2_ref, b2_ref, keys_ref, mkey_ref,
               a_ref):
    h2 = pl.program_id(0)
    s = pl.program_id(1)
    bs = x_ref.shape[0]

    @pl.when(h2 == 0)
    def _():
        a = lax.dot_general(x_ref[...], w1_ref[...], (((1,), (1,)), ((), ())),
                            preferred_element_type=jnp.float32)
        a_ref[pl.ds(s * bs, bs), :] = jnp.maximum(a + b1_ref[...], 0.0)

    sc = lax.dot_general(a_ref[pl.ds(s * bs, bs), :], w2_ref[...],
                         (((1,), (1,)), ((), ())),
                         preferred_element_type=jnp.float32)
    sc = sc + b2_ref[...]
    bits = lax.bitcast_convert_type(sc, jnp.int32)
    skey = jnp.where(bits >= 0, bits, bits ^ jnp.int32(_MONO_XOR))
    keys_ref[...] = skey
    cm = jnp.max(skey, axis=0, keepdims=True)
    prev = jnp.where(s == 0, jnp.int32(_INT_MIN), mkey_ref[...])
    mkey_ref[...] = jnp.maximum(prev, cm)


def _gate_keys(x, w1, b1_row, w2, b2_row, bs=256, bh=512):
    S, D = x.shape
    H = w1.shape[0]
    return pl.pallas_call(
        _gate_body,
        grid=(H // bh, S // bs),
        in_specs=[
            pl.BlockSpec((bs, D), lambda h, s: (s, 0)),
            pl.BlockSpec((H, D), lambda h, s: (0, 0)),
            pl.BlockSpec((1, H), lambda h, s: (0, 0)),
            pl.BlockSpec((bh, H), lambda h, s: (h, 0)),
            pl.BlockSpec((1, bh), lambda h, s: (0, h)),
        ],
        out_specs=[
            pl.BlockSpec((bs, bh), lambda h, s: (s, h)),
            pl.BlockSpec((1, bh), lambda h, s: (0, h)),
        ],
        out_shape=[
            jax.ShapeDtypeStruct((S, H), jnp.int32),
            jax.ShapeDtypeStruct((1, H), jnp.int32),
        ],
        scratch_shapes=[pltpu.VMEM((S, H), jnp.float32)],
    )(x, w1, b1_row, w2, b2_row)


_NLANES = 16
_WIN = 32768
_MROWS = 8
_UNROLL = 8


def _merge_and_scan(hist_hbm, st_in, tmp_v, mh_v, nbins, k, nw):
    nchunk = nbins // _NLANES

    def zbody(j, _):
        mh_v[pl.ds(j * _NLANES, _NLANES)] = jnp.zeros((_NLANES,), jnp.int32)
        return 0
    lax.fori_loop(0, nchunk, zbody, 0)

    def rbody(t, _):
        pltpu.sync_copy(hist_hbm.at[pl.ds(t * _MROWS, _MROWS)], tmp_v)

        def abody(j, _):
            sl = pl.ds(j * _NLANES, _NLANES)
            acc = mh_v[sl]
            for r in range(_MROWS):
                acc = acc + tmp_v[r, sl]
            mh_v[sl] = acc
            return 0
        lax.fori_loop(0, nchunk, abody, 0)
        return 0
    lax.fori_loop(0, nw // _MROWS, rbody, 0)

    if st_in is None:
        p_prev = jnp.int32(0)
        c_hi = jnp.int32(0)
    else:
        pltpu.sync_copy(st_in, tmp_v.at[0, pl.ds(0, 16)])
        stv = tmp_v[0, pl.ds(0, 16)]
        p_prev = stv[0]
        c_hi = stv[1]

    bsel, chi = _scan_top(mh_v, c_hi, k, nbins)
    return p_prev, c_hi, bsel, chi


def _scan_top(mh_v, c_hi, k, nbins):
    nchunk = nbins // _NLANES
    kk = jnp.int32(k)

    def cbody(j, carry):
        cum, csel, cums = carry
        c = jnp.int32(nchunk - 1) - j
        vchunk = mh_v[pl.ds(c * _NLANES, _NLANES)]
        ssum = jnp.sum(vchunk, axis=0)
        hit = jnp.logical_and(cum + ssum >= kk, csel < 0)
        csel = jnp.where(hit, c, csel)
        cums = jnp.where(hit, cum, cums)
        return (cum + ssum, csel, cums)

    _, csel, cum0 = lax.fori_loop(
        0, nchunk, cbody, (c_hi, jnp.int32(-1), jnp.int32(0)))

    vchunk = mh_v[pl.ds(csel * _NLANES, _NLANES)]
    lanes = lax.iota(jnp.int32, _NLANES)
    csum = plsc.cumsum(vchunk)
    total = jnp.sum(vchunk, axis=0)
    s_incl = cum0 + (total - csum) + vchunk
    hit = s_incl >= kk
    nhits = plsc.all_reduce_population_count(hit)
    bl = nhits[0] - 1
    bsel = csel * _NLANES + bl
    above = jnp.where(lanes > bl, vchunk, 0)
    chi = cum0 + jnp.sum(above, axis=0)
    return bsel, chi


def _hist_field(keys_hbm, win_v, sem0, sem1, lh_v, base, shift_bin, binmask,
                bin_off, shift_pred, pred_val, nbins, nwin):
    lanes = lax.iota(jnp.int32, _NLANES)
    lane_off = lanes * nbins
    ones = jnp.ones((_NLANES,), jnp.int32)

    zero = jnp.zeros((_NLANES,), jnp.int32)

    def zbody(j, _):
        for u in range(4):
            lh_v[pl.ds((j * 4 + u) * _NLANES, _NLANES)] = zero
        return 0
    lax.fori_loop(0, nbins // 4, zbody, 0)

    sems = [sem0, sem1]
    copies = [None, None]

    def start(w):
        b = w % 2
        copies[b] = pltpu.async_copy(
            keys_hbm.at[pl.ds(base + w * _WIN, _WIN)],
            win_v.at[pl.ds(b * _WIN, _WIN)], sems[b])

    def process(w):
        boff = (w % 2) * _WIN

        def ibody(i, _):
            ib = boff + i * (_NLANES * _UNROLL)
            vs = [win_v[pl.ds(ib + u * _NLANES, _NLANES)]
                  for u in range(_UNROLL)]
            for v in vs:
                bv = ((v >> shift_bin) & binmask) + bin_off
                addr = lane_off + bv
                if shift_pred is None:
                    plsc.addupdate_scatter(lh_v, [addr], ones)
                else:
                    pred = (v >> shift_pred) == pred_val
                    plsc.addupdate_scatter(lh_v, [addr], ones, mask=pred)
            return 0
        lax.fori_loop(0, _WIN // (_NLANES * _UNROLL), ibody, 0)

    start(0)
    for w in range(nwin):
        if w + 1 < nwin:
            start(w + 1)
        copies[w % 2].wait()
        process(w)


def _merge_lanes(lh_v, mh_v, nbins):
    def mbody(j, _):
        acc = jnp.zeros((_NLANES,), jnp.int32)
        for lane in range(_NLANES):
            acc = acc + lh_v[pl.ds(lane * nbins + j * _NLANES, _NLANES)]
        mh_v[pl.ds(j * _NLANES, _NLANES)] = acc
        return 0
    lax.fori_loop(0, nbins // _NLANES, mbody, 0)


def _radix_select(keys_flat, k):
    N = keys_flat.shape[0]
    nc, ns = 2, 16
    nw = nc * ns
    chunk = N // nw
    nwin = chunk // _WIN
    assert chunk % _WIN == 0 and N % nw == 0
    mesh = plsc.VectorSubcoreMesh(core_axis_name="c", subcore_axis_name="s",
                                  num_cores=nc, num_subcores=ns)

    def wid():
        return lax.axis_index("s") * nc + lax.axis_index("c")

    def scratches(nbins):
        return [
            pltpu.VMEM((2 * _WIN,), jnp.int32),
            pltpu.SemaphoreType.DMA,
            pltpu.SemaphoreType.DMA,
            pltpu.VMEM((_NLANES * 2048,), jnp.int32),
            pltpu.VMEM((2048,), jnp.int32),
            pltpu.VMEM((_MROWS, nbins), jnp.int32),
        ]

    @functools.partial(
        pl.kernel,
        out_type=jax.ShapeDtypeStruct((nw, 2048), jnp.int32),
        mesh=mesh,
        compiler_params=pltpu.CompilerParams(needs_layout_passes=False),
        scratch_types=scratches(2048))
    def l1(keys_hbm, hist_hbm, win_v, sem0, sem1, lh_v, mh_v, tmp_v):
        w = wid()
        _hist_field(keys_hbm, win_v, sem0, sem1, lh_v, w * chunk,
                    jnp.int32(21), jnp.int32(2047), jnp.int32(1024),
                    None, None, 2048, nwin)
        _merge_lanes(lh_v, mh_v, 2048)
        pltpu.sync_copy(mh_v, hist_hbm.at[w])

    chunk16 = N // ns
    nwin16 = chunk16 // _WIN

    @functools.partial(
        pl.kernel,
        out_type=[jax.ShapeDtypeStruct((nw, 1024), jnp.int32),
                  jax.ShapeDtypeStruct((16,), jnp.int32)],
        mesh=mesh,
        compiler_params=pltpu.CompilerParams(needs_layout_passes=False),
        scratch_types=scratches(2048)
        + [pltpu.VMEM_SHARED((ns, 2048), jnp.int32)])
    def l23(keys_hbm, h1_hbm, hist_hbm, st_hbm, win_v, sem0, sem1, lh_v,
            mh_v, tmp_v, stage_sh):
        sid = lax.axis_index("s")
        w = wid()
        _, _, bsel, chi1 = _merge_and_scan(h1_hbm, None, tmp_v, mh_v, 2048,
                                           k, nw)
        p1 = bsel - jnp.int32(1024)

        _hist_field(keys_hbm, win_v, sem0, sem1, lh_v, sid * chunk16,
                    jnp.int32(10), jnp.int32(2047), jnp.int32(0),
                    jnp.int32(21), p1, 2048, nwin16)
        _merge_lanes(lh_v, mh_v, 2048)
        pltpu.sync_copy(mh_v, stage_sh.at[sid])
        plsc.subcore_barrier()

        def zb(j, _):
            mh_v[pl.ds(j * _NLANES, _NLANES)] = jnp.zeros((_NLANES,),
                                                          jnp.int32)
            return 0
        lax.fori_loop(0, 2048 // _NLANES, zb, 0)

        def rb(t, _):
            pltpu.sync_copy(stage_sh.at[t], tmp_v.at[0])

            def ab(j, _):
                sl = pl.ds(j * _NLANES, _NLANES)
                mh_v[sl] = mh_v[sl] + tmp_v[0, sl]
                return 0
            lax.fori_loop(0, 2048 // _NLANES, ab, 0)
            return 0
        lax.fori_loop(0, ns, rb, 0)

        bsel2, chi2 = _scan_top(mh_v, chi1, k, 2048)
        p2 = p1 * jnp.int32(2048) + bsel2

        _hist_field(keys_hbm, win_v, sem0, sem1, lh_v, w * chunk,
                    jnp.int32(0), jnp.int32(1023), jnp.int32(0),
                    jnp.int32(10), p2, 1024, nwin)
        _merge_lanes(lh_v, mh_v, 1024)
        pltpu.sync_copy(mh_v.at[pl.ds(0, 1024)], hist_hbm.at[w])

        @pl.when(w == 0)
        def _():
            lanes = lax.iota(jnp.int32, 16)
            st = jnp.where(lanes == 0, p2, jnp.where(lanes == 1, chi2, 0))
            tmp_v[0, pl.ds(0, 16)] = st
            pltpu.sync_copy(tmp_v.at[0, pl.ds(0, 16)], st_hbm)

    h1 = l1(keys_flat)
    h3, st2 = l23(keys_flat, h1)
    return h3, st2


def _wsel_body(up_ref, mod_ref, mcol_ref, h3_ref, st_ref, k_ref, w_ref,
               thr_ref):
    nb = h3_ref.shape[1]
    hist = jnp.sum(h3_ref[...].astype(jnp.float32), axis=0, keepdims=True)
    bi = lax.broadcasted_iota(jnp.int32, (nb, nb), 0)
    bj = lax.broadcasted_iota(jnp.int32, (nb, nb), 1)
    lower = jnp.where(bi >= bj, 1.0, 0.0)
    suffix = lax.dot_general(hist, lower, (((1,), (0,)), ((), ())),
                             preferred_element_type=jnp.float32)
    stv = st_ref[...]
    p2 = stv[0, 0]
    c_hi = stv[0, 1].astype(jnp.float32)
    kk = k_ref[...][0, 0].astype(jnp.float32)
    hits = jnp.sum(jnp.where(suffix + c_hi >= kk, 1, 0))
    bsel = hits.astype(jnp.int32) - 1
    thr = p2 * jnp.int32(nb) + bsel
    thr_ref[...] = jnp.zeros_like(thr_ref) + thr
    mask = mcol_ref[...] >= thr
    w_ref[...] = jnp.where(mask, mod_ref[...], up_ref[...])


def _wsel(up_w, mod_w, mkey_col, h3, st2, k, bh=384):
    H, D = up_w.shape
    nw, nb = h3.shape
    karr = jnp.full((1, 16), k, jnp.int32)
    return pl.pallas_call(
        _wsel_body,
        grid=(H // bh,),
        in_specs=[
            pl.BlockSpec((bh, D), lambda h: (h, 0)),
            pl.BlockSpec((bh, D), lambda h: (h, 0)),
            pl.BlockSpec((bh, 1), lambda h: (h, 0)),
            pl.BlockSpec((nw, nb), lambda h: (0, 0)),
            pl.BlockSpec((1, 16), lambda h: (0, 0)),
            pl.BlockSpec((1, 16), lambda h: (0, 0)),
        ],
        out_specs=[
            pl.BlockSpec((bh, D), lambda h: (h, 0)),
            pl.BlockSpec((1, 16), lambda h: (0, 0)),
        ],
        out_shape=[
            jax.ShapeDtypeStruct((H, D), jnp.float32),
            jax.ShapeDtypeStruct((1, 16), jnp.int32),
        ],
    )(up_w, mod_w, mkey_col, h3, st2.reshape(1, 16), karr)


def _final_body(x_ref, w_ref, dw_ref, mrow_ref, thr_ref, upb_ref, modb_ref,
                db_ref, o_ref):
    h = lax.dot_general(x_ref[...], w_ref[...], (((1,), (1,)), ((), ())),
                        preferred_element_type=jnp.float32)
    bsel = jnp.where(mrow_ref[...] >= thr_ref[...][0, 0], modb_ref[...],
                     upb_ref[...])
    h = h + bsel
    g = 0.5 * h * (1.0 + lax.erf(h * 0.7071067811865476))
    out = lax.dot_general(g, dw_ref[...], (((1,), (1,)), ((), ())),
                          preferred_element_type=jnp.float32)
    o_ref[...] = out + db_ref[...]


def _final(x, w_sel, down_w, mkey_row, thr_row, up_b_row, mod_b_row,
           down_b_row, bs=256):
    S, D = x.shape
    H = w_sel.shape[0]
    return pl.pallas_call(
        _final_body,
        grid=(S // bs,),
        in_specs=[
            pl.BlockSpec((bs, D), lambda s: (s, 0)),
            pl.BlockSpec((H, D), lambda s: (0, 0)),
            pl.BlockSpec((D, H), lambda s: (0, 0)),
            pl.BlockSpec((1, H), lambda s: (0, 0)),
            pl.BlockSpec((1, 16), lambda s: (0, 0)),
            pl.BlockSpec((1, H), lambda s: (0, 0)),
            pl.BlockSpec((1, H), lambda s: (0, 0)),
            pl.BlockSpec((1, D), lambda s: (0, 0)),
        ],
        out_specs=pl.BlockSpec((bs, D), lambda s: (s, 0)),
        out_shape=jax.ShapeDtypeStruct((S, D), jnp.float32),
    )(x, w_sel, down_w, mkey_row, thr_row, up_b_row, mod_b_row, down_b_row)


def kernel(inputs, up_w, up_b, gate_w1, gate_b1, gate_w2, gate_b2,
           mod_w, mod_b, down_w, down_b):
    B, S, D = inputs.shape
    H = up_w.shape[0]
    k = min(_TOP_K_PER_ROW * S, S * H)
    x = inputs.reshape(S, D)

    keys, mkey = _gate_keys(x, gate_w1, gate_b1.reshape(1, H), gate_w2,
                            gate_b2.reshape(1, H))
    h3, st2 = _radix_select(keys.reshape(S * H), k)
    h3m = jnp.sum(h3, axis=0, keepdims=True)
    w_sel, thr_row = _wsel(up_w, mod_w, mkey.reshape(H, 1), h3m, st2, k)
    out = _final(x, w_sel, down_w, mkey, thr_row, up_b.reshape(1, H),
                 mod_b.reshape(1, H), down_b.reshape(1, D))
    return out.reshape(B, S, D)


# speedup vs baseline: 1.2266x; 1.0618x over previous
import functools

import jax
import jax.numpy as jnp
from jax import lax
from jax.experimental import pallas as pl
from jax.experimental.pallas import tpu as pltpu
from jax.experimental.pallas import tpu_sc as plsc

_TOP_K_PER_ROW = 8

_INT_MIN = -(2**31)
_MONO_XOR = 0x7FFFFFFF


def _gate_body(x_ref, w1_ref, b1_ref, w2_ref, b2_ref, keys_ref, mkey_ref,
               a_ref):
    h2 = pl.program_id(0)
    s = pl.program_id(1)
    bs = x_ref.shape[0]

    @pl.when(h2 == 0)
    def _():
        a = lax.dot_general(x_ref[...], w1_ref[...], (((1,), (1,)), ((), ())),
                            preferred_element_type=jnp.float32)
        a_ref[pl.ds(s * bs, bs), :] = jnp.maximum(a + b1_ref[...], 0.0)

    sc = lax.dot_general(a_ref[pl.ds(s * bs, bs), :], w2_ref[...],
                         (((1,), (1,)), ((), ())),
                         preferred_element_type=jnp.float32)
    sc = sc + b2_ref[...]
    bits = lax.bitcast_convert_type(sc, jnp.int32)
    skey = jnp.where(bits >= 0, bits, bits ^ jnp.int32(_MONO_XOR))
    keys_ref[...] = skey
    cm = jnp.max(skey, axis=0, keepdims=True)
    prev = jnp.where(s == 0, jnp.int32(_INT_MIN), mkey_ref[...])
    mkey_ref[...] = jnp.maximum(prev, cm)


def _gate_keys(x, w1, b1_row, w2, b2_row, bs=256, bh=512):
    S, D = x.shape
    H = w1.shape[0]
    return pl.pallas_call(
        _gate_body,
        grid=(H // bh, S // bs),
        in_specs=[
            pl.BlockSpec((bs, D), lambda h, s: (s, 0)),
            pl.BlockSpec((H, D), lambda h, s: (0, 0)),
            pl.BlockSpec((1, H), lambda h, s: (0, 0)),
            pl.BlockSpec((bh, H), lambda h, s: (h, 0)),
            pl.BlockSpec((1, bh), lambda h, s: (0, h)),
        ],
        out_specs=[
            pl.BlockSpec((bs, bh), lambda h, s: (s, h)),
            pl.BlockSpec((1, bh), lambda h, s: (0, h)),
        ],
        out_shape=[
            jax.ShapeDtypeStruct((S, H), jnp.int32),
            jax.ShapeDtypeStruct((1, H), jnp.int32),
        ],
        scratch_shapes=[pltpu.VMEM((S, H), jnp.float32)],
    )(x, w1, b1_row, w2, b2_row)


_NLANES = 16
_WIN = 32768
_MROWS = 8
_UNROLL = 8


def _merge_and_scan(hist_hbm, st_in, tmp_v, mh_v, nbins, k, nw):
    nchunk = nbins // _NLANES

    def zbody(j, _):
        mh_v[pl.ds(j * _NLANES, _NLANES)] = jnp.zeros((_NLANES,), jnp.int32)
        return 0
    lax.fori_loop(0, nchunk, zbody, 0)

    def rbody(t, _):
        pltpu.sync_copy(hist_hbm.at[pl.ds(t * _MROWS, _MROWS)], tmp_v)

        def abody(j, _):
            sl = pl.ds(j * _NLANES, _NLANES)
            acc = mh_v[sl]
            for r in range(_MROWS):
                acc = acc + tmp_v[r, sl]
            mh_v[sl] = acc
            return 0
        lax.fori_loop(0, nchunk, abody, 0)
        return 0
    lax.fori_loop(0, nw // _MROWS, rbody, 0)

    if st_in is None:
        p_prev = jnp.int32(0)
        c_hi = jnp.int32(0)
    else:
        pltpu.sync_copy(st_in, tmp_v.at[0, pl.ds(0, 16)])
        stv = tmp_v[0, pl.ds(0, 16)]
        p_prev = stv[0]
        c_hi = stv[1]

    kk = jnp.int32(k)

    def cbody(j, carry):
        cum, csel, cums = carry
        c = jnp.int32(nchunk - 1) - j
        vchunk = mh_v[pl.ds(c * _NLANES, _NLANES)]
        ssum = jnp.sum(vchunk, axis=0)
        hit = jnp.logical_and(cum + ssum >= kk, csel < 0)
        csel = jnp.where(hit, c, csel)
        cums = jnp.where(hit, cum, cums)
        return (cum + ssum, csel, cums)

    _, csel, cum0 = lax.fori_loop(
        0, nchunk, cbody, (c_hi, jnp.int32(-1), jnp.int32(0)))

    vchunk = mh_v[pl.ds(csel * _NLANES, _NLANES)]
    lanes = lax.iota(jnp.int32, _NLANES)
    csum = plsc.cumsum(vchunk)
    total = jnp.sum(vchunk, axis=0)
    s_incl = cum0 + (total - csum) + vchunk
    hit = s_incl >= kk
    nhits = plsc.all_reduce_population_count(hit)
    bl = nhits[0] - 1
    bsel = csel * _NLANES + bl
    above = jnp.where(lanes > bl, vchunk, 0)
    chi = cum0 + jnp.sum(above, axis=0)
    return p_prev, c_hi, bsel, chi


def _hist_field(keys_hbm, win_v, sem0, sem1, lh_v, base, shift_bin, binmask,
                bin_off, shift_pred, pred_val, nbins, nwin):
    lanes = lax.iota(jnp.int32, _NLANES)
    lane_off = lanes * nbins
    ones = jnp.ones((_NLANES,), jnp.int32)

    zero = jnp.zeros((_NLANES,), jnp.int32)

    def zbody(j, _):
        for u in range(4):
            lh_v[pl.ds((j * 4 + u) * _NLANES, _NLANES)] = zero
        return 0
    lax.fori_loop(0, nbins // 4, zbody, 0)

    sems = [sem0, sem1]
    copies = [None, None]

    def start(w):
        b = w % 2
        copies[b] = pltpu.async_copy(
            keys_hbm.at[pl.ds(base + w * _WIN, _WIN)],
            win_v.at[pl.ds(b * _WIN, _WIN)], sems[b])

    def process(w):
        boff = (w % 2) * _WIN

        def ibody(i, _):
            ib = boff + i * (_NLANES * _UNROLL)
            vs = [win_v[pl.ds(ib + u * _NLANES, _NLANES)]
                  for u in range(_UNROLL)]
            for v in vs:
                bv = ((v >> shift_bin) & binmask) + bin_off
                addr = lane_off + bv
                if shift_pred is None:
                    plsc.addupdate_scatter(lh_v, [addr], ones)
                else:
                    pred = (v >> shift_pred) == pred_val
                    plsc.addupdate_scatter(lh_v, [addr], ones, mask=pred)
            return 0
        lax.fori_loop(0, _WIN // (_NLANES * _UNROLL), ibody, 0)

    start(0)
    for w in range(nwin):
        if w + 1 < nwin:
            start(w + 1)
        copies[w % 2].wait()
        process(w)


def _merge_lanes(lh_v, mh_v, nbins):
    def mbody(j, _):
        acc = jnp.zeros((_NLANES,), jnp.int32)
        for lane in range(_NLANES):
            acc = acc + lh_v[pl.ds(lane * nbins + j * _NLANES, _NLANES)]
        mh_v[pl.ds(j * _NLANES, _NLANES)] = acc
        return 0
    lax.fori_loop(0, nbins // _NLANES, mbody, 0)


def _radix_select(keys_flat, k):
    N = keys_flat.shape[0]
    nc, ns = 2, 16
    nw = nc * ns
    chunk = N // nw
    nwin = chunk // _WIN
    assert chunk % _WIN == 0 and N % nw == 0
    mesh = plsc.VectorSubcoreMesh(core_axis_name="c", subcore_axis_name="s",
                                  num_cores=nc, num_subcores=ns)

    def wid():
        return lax.axis_index("s") * nc + lax.axis_index("c")

    def scratches(nbins):
        return [
            pltpu.VMEM((2 * _WIN,), jnp.int32),
            pltpu.SemaphoreType.DMA,
            pltpu.SemaphoreType.DMA,
            pltpu.VMEM((_NLANES * 2048,), jnp.int32),
            pltpu.VMEM((2048,), jnp.int32),
            pltpu.VMEM((_MROWS, nbins), jnp.int32),
        ]

    @functools.partial(
        pl.kernel,
        out_type=jax.ShapeDtypeStruct((nw, 2048), jnp.int32),
        mesh=mesh,
        compiler_params=pltpu.CompilerParams(needs_layout_passes=False),
        scratch_types=scratches(2048))
    def l1(keys_hbm, hist_hbm, win_v, sem0, sem1, lh_v, mh_v, tmp_v):
        w = wid()
        _hist_field(keys_hbm, win_v, sem0, sem1, lh_v, w * chunk,
                    jnp.int32(21), jnp.int32(2047), jnp.int32(1024),
                    None, None, 2048, nwin)
        _merge_lanes(lh_v, mh_v, 2048)
        pltpu.sync_copy(mh_v, hist_hbm.at[w])

    @functools.partial(
        pl.kernel,
        out_type=[jax.ShapeDtypeStruct((nw, 2048), jnp.int32),
                  jax.ShapeDtypeStruct((16,), jnp.int32)],
        mesh=mesh,
        compiler_params=pltpu.CompilerParams(needs_layout_passes=False),
        scratch_types=scratches(2048))
    def l2(keys_hbm, h1_hbm, hist_hbm, st_hbm, win_v, sem0, sem1, lh_v, mh_v, tmp_v):
        w = wid()
        _, _, bsel, chi = _merge_and_scan(h1_hbm, None, tmp_v, mh_v, 2048, k, nw)
        p1 = bsel - jnp.int32(1024)
        _hist_field(keys_hbm, win_v, sem0, sem1, lh_v, w * chunk,
                    jnp.int32(10), jnp.int32(2047), jnp.int32(0),
                    jnp.int32(21), p1, 2048, nwin)
        _merge_lanes(lh_v, mh_v, 2048)
        pltpu.sync_copy(mh_v, hist_hbm.at[w])

        @pl.when(w == 0)
        def _():
            lanes = lax.iota(jnp.int32, 16)
            st = jnp.where(lanes == 0, p1, jnp.where(lanes == 1, chi, 0))
            tmp_v[0, pl.ds(0, 16)] = st
            pltpu.sync_copy(tmp_v.at[0, pl.ds(0, 16)], st_hbm)

    @functools.partial(
        pl.kernel,
        out_type=[jax.ShapeDtypeStruct((nw, 1024), jnp.int32),
                  jax.ShapeDtypeStruct((16,), jnp.int32)],
        mesh=mesh,
        compiler_params=pltpu.CompilerParams(needs_layout_passes=False),
        scratch_types=scratches(2048))
    def l3(keys_hbm, h2_hbm, st1_hbm, hist_hbm, st_hbm, win_v, sem0, sem1, lh_v, mh_v, tmp_v):
        w = wid()
        p1, _, bsel, chi = _merge_and_scan(h2_hbm, st1_hbm, tmp_v, mh_v, 2048, k, nw)
        p2 = p1 * jnp.int32(2048) + bsel
        _hist_field(keys_hbm, win_v, sem0, sem1, lh_v, w * chunk,
                    jnp.int32(0), jnp.int32(1023), jnp.int32(0),
                    jnp.int32(10), p2, 1024, nwin)
        _merge_lanes(lh_v, mh_v, 1024)
        pltpu.sync_copy(mh_v.at[pl.ds(0, 1024)], hist_hbm.at[w])

        @pl.when(w == 0)
        def _():
            lanes = lax.iota(jnp.int32, 16)
            st = jnp.where(lanes == 0, p2, jnp.where(lanes == 1, chi, 0))
            tmp_v[0, pl.ds(0, 16)] = st
            pltpu.sync_copy(tmp_v.at[0, pl.ds(0, 16)], st_hbm)

    h1 = l1(keys_flat)
    h2, st1 = l2(keys_flat, h1)
    h3, st2 = l3(keys_flat, h2, st1)
    return h3, st2


def _wsel_body(up_ref, mod_ref, mcol_ref, h3_ref, st_ref, k_ref, w_ref,
               thr_ref):
    nb = h3_ref.shape[1]
    hist = jnp.sum(h3_ref[...].astype(jnp.float32), axis=0, keepdims=True)
    bi = lax.broadcasted_iota(jnp.int32, (nb, nb), 0)
    bj = lax.broadcasted_iota(jnp.int32, (nb, nb), 1)
    lower = jnp.where(bi >= bj, 1.0, 0.0)
    suffix = lax.dot_general(hist, lower, (((1,), (0,)), ((), ())),
                             preferred_element_type=jnp.float32)
    stv = st_ref[...]
    p2 = stv[0, 0]
    c_hi = stv[0, 1].astype(jnp.float32)
    kk = k_ref[...][0, 0].astype(jnp.float32)
    hits = jnp.sum(jnp.where(suffix + c_hi >= kk, 1, 0))
    bsel = hits.astype(jnp.int32) - 1
    thr = p2 * jnp.int32(nb) + bsel
    thr_ref[...] = jnp.zeros_like(thr_ref) + thr
    mask = mcol_ref[...] >= thr
    w_ref[...] = jnp.where(mask, mod_ref[...], up_ref[...])


def _wsel(up_w, mod_w, mkey_col, h3, st2, k, bh=384):
    H, D = up_w.shape
    nw, nb = h3.shape
    karr = jnp.full((1, 16), k, jnp.int32)
    return pl.pallas_call(
        _wsel_body,
        grid=(H // bh,),
        in_specs=[
            pl.BlockSpec((bh, D), lambda h: (h, 0)),
            pl.BlockSpec((bh, D), lambda h: (h, 0)),
            pl.BlockSpec((bh, 1), lambda h: (h, 0)),
            pl.BlockSpec((nw, nb), lambda h: (0, 0)),
            pl.BlockSpec((1, 16), lambda h: (0, 0)),
            pl.BlockSpec((1, 16), lambda h: (0, 0)),
        ],
        out_specs=[
            pl.BlockSpec((bh, D), lambda h: (h, 0)),
            pl.BlockSpec((1, 16), lambda h: (0, 0)),
        ],
        out_shape=[
            jax.ShapeDtypeStruct((H, D), jnp.float32),
            jax.ShapeDtypeStruct((1, 16), jnp.int32),
        ],
    )(up_w, mod_w, mkey_col, h3, st2.reshape(1, 16), karr)


def _final_body(x_ref, w_ref, dw_ref, mrow_ref, thr_ref, upb_ref, modb_ref,
                db_ref, o_ref):
    h = lax.dot_general(x_ref[...], w_ref[...], (((1,), (1,)), ((), ())),
                        preferred_element_type=jnp.float32)
    bsel = jnp.where(mrow_ref[...] >= thr_ref[...][0, 0], modb_ref[...],
                     upb_ref[...])
    h = h + bsel
    g = 0.5 * h * (1.0 + lax.erf(h * 0.7071067811865476))
    out = lax.dot_general(g, dw_ref[...], (((1,), (1,)), ((), ())),
                          preferred_element_type=jnp.float32)
    o_ref[...] = out + db_ref[...]


def _final(x, w_sel, down_w, mkey_row, thr_row, up_b_row, mod_b_row,
           down_b_row, bs=256):
    S, D = x.shape
    H = w_sel.shape[0]
    return pl.pallas_call(
        _final_body,
        grid=(S // bs,),
        in_specs=[
            pl.BlockSpec((bs, D), lambda s: (s, 0)),
            pl.BlockSpec((H, D), lambda s: (0, 0)),
            pl.BlockSpec((D, H), lambda s: (0, 0)),
            pl.BlockSpec((1, H), lambda s: (0, 0)),
            pl.BlockSpec((1, 16), lambda s: (0, 0)),
            pl.BlockSpec((1, H), lambda s: (0, 0)),
            pl.BlockSpec((1, H), lambda s: (0, 0)),
            pl.BlockSpec((1, D), lambda s: (0, 0)),
        ],
        out_specs=pl.BlockSpec((bs, D), lambda s: (s, 0)),
        out_shape=jax.ShapeDtypeStruct((S, D), jnp.float32),
    )(x, w_sel, down_w, mkey_row, thr_row, up_b_row, mod_b_row, down_b_row)


def kernel(inputs, up_w, up_b, gate_w1, gate_b1, gate_w2, gate_b2,
           mod_w, mod_b, down_w, down_b):
    B, S, D = inputs.shape
    H = up_w.shape[0]
    k = min(_TOP_K_PER_ROW * S, S * H)
    x = inputs.reshape(S, D)

    keys, mkey = _gate_keys(x, gate_w1, gate_b1.reshape(1, H), gate_w2,
                            gate_b2.reshape(1, H))
    h3, st2 = _radix_select(keys.reshape(S * H), k)
    h3m = jnp.sum(h3, axis=0, keepdims=True)
    w_sel, thr_row = _wsel(up_w, mod_w, mkey.reshape(H, 1), h3m, st2, k)
    out = _final(x, w_sel, down_w, mkey, thr_row, up_b.reshape(1, H),
                 mod_b.reshape(1, H), down_b.reshape(1, D))
    return out.reshape(B, S, D)
